# baseline (speedup 1.0000x reference)
import functools

import jax
import jax.numpy as jnp
from jax import lax
from jax.experimental import pallas as pl
from jax.experimental.pallas import tpu as pltpu
from jax.experimental.pallas import tpu_sc as plsc

N = 10000
D = 128
NC = 2
NS = 16
NW = NC * NS
CHUNK = 128
WIN = 6
N_PAD = 10240
ROWS_PER_SUB = N_PAD // NS
NB = N_PAD // D


def _sc_segment_sum(x, edg, base, extra):
    nwin = base // WIN
    mesh = plsc.VectorSubcoreMesh(
        core_axis_name="c", subcore_axis_name="s", num_cores=NC,
        num_subcores=NS)

    @functools.partial(
        pl.kernel,
        out_type=(
            jax.ShapeDtypeStruct((NC, N_PAD, D), jnp.float32),
            jax.ShapeDtypeStruct((NC, N_PAD), jnp.float32),
        ),
        mesh=mesh,
        scratch_types=[
            pltpu.VMEM((WIN, 2, CHUNK), jnp.int32),
            pltpu.VMEM((WIN, 2, CHUNK), jnp.int32),
            pltpu.VMEM((CHUNK, D), jnp.float32),
            pltpu.VMEM((CHUNK, D), jnp.float32),
            pltpu.VMEM((CHUNK,), jnp.float32),
            pltpu.VMEM((CHUNK,), jnp.float32),
            pltpu.VMEM_SHARED((N_PAD, D), jnp.float32),
            pltpu.VMEM_SHARED((N_PAD,), jnp.float32),
            pltpu.SemaphoreType.DMA,
            pltpu.SemaphoreType.DMA,
            pltpu.SemaphoreType.DMA,
            pltpu.SemaphoreType.DMA,
            pltpu.SemaphoreType.DMA,
        ],
    )
    def k(x_hbm, edg_hbm, agg_out, deg_out,
          idx0_v, idx1_v, rows0_v, rows1_v, ones_v, zed_v,
          agg_sh, deg_sh, sem0, sem1, semi0, semi1, ssem):
        idxb = (idx0_v, idx1_v)
        rows = (rows1_v, rows0_v)
        sems = (sem1, sem0)
        semi = (semi0, semi1)
        c = lax.axis_index("c")
        s = lax.axis_index("s")
        wid = c * NS + s
        row0 = s * ROWS_PER_SUB
        c0 = wid * base + jnp.minimum(wid, extra)

        pltpu.sync_copy(edg_hbm.at[pl.ds(c0, WIN)], idx0_v)
        pltpu.async_copy(x_hbm.at[idx0_v.at[0, 0]], rows[0], sems[0])

        z16 = jnp.zeros((16,), jnp.float32)
        o16 = jnp.ones((16,), jnp.float32)
        for j in range(CHUNK // 16):
            ones_v[pl.ds(j * 16, 16)] = o16
            zed_v[pl.ds(j * 16, 16)] = z16

        def zrow(i, carry):
            for j in range(D // 16):
                rows0_v[i, pl.ds(j * 16, 16)] = z16
            return carry
        lax.fori_loop(0, CHUNK, zrow, 0)

        for kk in range(ROWS_PER_SUB // CHUNK):
            pltpu.sync_copy(rows0_v, agg_sh.at[pl.ds(row0 + kk * CHUNK, CHUNK)])
            pltpu.sync_copy(zed_v, deg_sh.at[pl.ds(row0 + kk * CHUNK, CHUNK)])
        plsc.subcore_barrier()

        def window(w, q):
            @pl.when(w + 1 < nwin)
            def _():
                pltpu.async_copy(edg_hbm.at[pl.ds(c0 + (w + 1) * WIN, WIN)],
                                 idxb[1 - q], semi[1 - q])

            def body(p, carry):
                for t in range(2):
                    lj = p * 2 + t
                    nxt = lj + 1

                    @pl.when(w * WIN + lj > 0)
                    def _():
                        pltpu.make_async_copy(
                            rows[1 - t], agg_sh.at[idxb[q].at[lj, 1]],
                            ssem).wait()

                    @pl.when(nxt < WIN)
                    def _():
                        pltpu.async_copy(x_hbm.at[idxb[q].at[nxt, 0]],
                                         rows[1 - t], sems[1 - t])

                    @pl.when(jnp.logical_and(nxt == WIN, w + 1 < nwin))
                    def _():
                        pltpu.make_async_copy(
                            edg_hbm.at[pl.ds(c0 + (w + 1) * WIN, WIN)],
                            idxb[1 - q], semi[1 - q]).wait()
                        pltpu.async_copy(x_hbm.at[idxb[1 - q].at[0, 0]],
                                         rows[1 - t], sems[1 - t])

                    pltpu.make_async_copy(x_hbm.at[idxb[q].at[lj, 0]],
                                          rows[t], sems[t]).wait()
                    pltpu.async_copy(rows[t], agg_sh.at[idxb[q].at[lj, 1]],
                                     ssem, add=True)
                    pltpu.sync_copy(ones_v, deg_sh.at[idxb[q].at[lj, 1]],
                                    add=True)
                return carry
            lax.fori_loop(0, WIN // 2, body, 0)

        def wpair(w2, carry):
            window(w2 * 2, 0)
            window(w2 * 2 + 1, 1)
            return carry
        lax.fori_loop(0, nwin // 2, wpair, 0)
        if nwin % 2:
            window(nwin - 1, 0)
        pltpu.make_async_copy(rows[(base - 1) % 2],
                              agg_sh.at[idxb[0].at[0, 1]], ssem).wait()

        if extra:
            @pl.when(wid < extra)
            def _():
                tq = nwin % 2
                pltpu.sync_copy(edg_hbm.at[pl.ds(c0 + base, 1)],
                                idxb[tq].at[pl.ds(0, 1)])
                pltpu.async_copy(x_hbm.at[idxb[tq].at[0, 0]],
                                 rows0_v, sem0).wait()
                pltpu.sync_copy(rows0_v, agg_sh.at[idxb[tq].at[0, 1]],
                                add=True)
                pltpu.sync_copy(ones_v, deg_sh.at[idxb[tq].at[0, 1]],
                                add=True)

        plsc.subcore_barrier()

        pltpu.sync_copy(agg_sh.at[pl.ds(row0, ROWS_PER_SUB)],
                        agg_out.at[c, pl.ds(row0, ROWS_PER_SUB)])
        pltpu.sync_copy(deg_sh.at[pl.ds(row0, ROWS_PER_SUB)],
                        deg_out.at[c, pl.ds(row0, ROWS_PER_SUB)])

    return k(x, edg)


def _tc_finale(agg, degb, x, W, b, gamma, beta):
    def body(agg_ref, deg_ref, x_ref, w_ref, b_ref, g_ref, be_ref, o_ref):
        av = agg_ref[...]
        a = av[0] + av[1]
        db = deg_ref[...]
        recip = 1.0 / jnp.maximum(db[:, 0] + db[:, 1], 1.0)
        rt = jnp.transpose(recip)
        m = jnp.concatenate(
            [a[i * 128:(i + 1) * 128] * lax.slice(rt, (0, i), (128, i + 1))
             for i in range(NB)], axis=0)
        o = jnp.dot(m, w_ref[...], preferred_element_type=jnp.float32)
        o = o[:N] + b_ref[...]
        mu = jnp.mean(o, axis=0, keepdims=True)
        var = jnp.mean((o - mu) * (o - mu), axis=0, keepdims=True)
        o = (o - mu) * lax.rsqrt(var + 1e-5) * g_ref[...] + be_ref[...]
        o = jnp.where(o > 0.0, o, jnp.exp(jnp.minimum(o, 0.0)) - 1.0)
        o_ref[...] = x_ref[...] + o

    return pl.pallas_call(
        body,
        out_shape=jax.ShapeDtypeStruct((N, D), jnp.float32),
    )(agg, degb, x, W, b, gamma, beta)


def kernel(x, edges, W, b, gamma, beta):
    E = edges.shape[1]
    assert E % CHUNK == 0
    nchunks = E // CHUNK
    base = nchunks // NW
    base -= base % WIN
    extra = nchunks - base * NW
    assert 0 <= extra < NW or base == 0

    edg = edges.astype(jnp.int32).reshape(2, nchunks, CHUNK)
    edg = edg.transpose(1, 0, 2)

    agg, deg = _sc_segment_sum(x, edg, base, extra)

    degb = deg.reshape(NC, NB, 128).transpose(1, 0, 2)

    return _tc_finale(
        agg, degb, x, W,
        b.reshape(1, D), gamma.reshape(1, D), beta.reshape(1, D),
    )


# speedup vs baseline: 1.0199x; 1.0199x over previous
import functools

import jax
import jax.numpy as jnp
from jax import lax
from jax.experimental import pallas as pl
from jax.experimental.pallas import tpu as pltpu
from jax.experimental.pallas import tpu_sc as plsc

N = 10000
D = 128
NC = 2
NS = 16
NW = NC * NS
CHUNK = 128
WIN = 6
N_PAD = 10240
ROWS_PER_SUB = N_PAD // NS
NB = N_PAD // D


def _sc_segment_sum(x, edg, base, extra):
    nwin = base // WIN
    mesh = plsc.VectorSubcoreMesh(
        core_axis_name="c", subcore_axis_name="s", num_cores=NC,
        num_subcores=NS)

    @functools.partial(
        pl.kernel,
        out_type=(
            jax.ShapeDtypeStruct((NC, N_PAD, D), jnp.float32),
            jax.ShapeDtypeStruct((NC, N_PAD), jnp.float32),
        ),
        mesh=mesh,
        scratch_types=[
            pltpu.VMEM((WIN, 2, CHUNK), jnp.int32),
            pltpu.VMEM((WIN, 2, CHUNK), jnp.int32),
            pltpu.VMEM((CHUNK, D), jnp.float32),
            pltpu.VMEM((CHUNK, D), jnp.float32),
            pltpu.VMEM((CHUNK,), jnp.float32),
            pltpu.VMEM((CHUNK,), jnp.float32),
            pltpu.VMEM_SHARED((N_PAD, D), jnp.float32),
            pltpu.VMEM_SHARED((N_PAD,), jnp.float32),
            pltpu.SemaphoreType.DMA,
            pltpu.SemaphoreType.DMA,
            pltpu.SemaphoreType.DMA,
            pltpu.SemaphoreType.DMA,
        ],
    )
    def k(x_hbm, edg_hbm, agg_out, deg_out,
          idx0_v, idx1_v, rows0_v, rows1_v, ones_v, zed_v,
          agg_sh, deg_sh, sem0, sem1, semi0, semi1):
        idxb = (idx0_v, idx1_v)
        rows = (rows1_v, rows0_v)
        sems = (sem1, sem0)
        semi = (semi0, semi1)
        c = lax.axis_index("c")
        s = lax.axis_index("s")
        wid = c * NS + s
        row0 = s * ROWS_PER_SUB
        c0 = wid * base + jnp.minimum(wid, extra)

        pltpu.sync_copy(edg_hbm.at[pl.ds(c0, WIN)], idx0_v)
        pltpu.async_copy(x_hbm.at[idx0_v.at[0, 0]], rows[0], sems[0])

        z16 = jnp.zeros((16,), jnp.float32)
        o16 = jnp.ones((16,), jnp.float32)
        for j in range(CHUNK // 16):
            ones_v[pl.ds(j * 16, 16)] = o16
            zed_v[pl.ds(j * 16, 16)] = z16

        def zrow(i, carry):
            for j in range(D // 16):
                rows0_v[i, pl.ds(j * 16, 16)] = z16
            return carry
        lax.fori_loop(0, CHUNK, zrow, 0)

        for kk in range(ROWS_PER_SUB // CHUNK):
            pltpu.sync_copy(rows0_v, agg_sh.at[pl.ds(row0 + kk * CHUNK, CHUNK)])
            pltpu.sync_copy(zed_v, deg_sh.at[pl.ds(row0 + kk * CHUNK, CHUNK)])
        plsc.subcore_barrier()

        def window(w, q):
            @pl.when(w + 1 < nwin)
            def _():
                pltpu.async_copy(edg_hbm.at[pl.ds(c0 + (w + 1) * WIN, WIN)],
                                 idxb[1 - q], semi[1 - q])

            def body(p, carry):
                for t in range(2):
                    lj = p * 2 + t
                    nxt = lj + 1

                    @pl.when(nxt < WIN)
                    def _():
                        pltpu.async_copy(x_hbm.at[idxb[q].at[nxt, 0]],
                                         rows[1 - t], sems[1 - t])

                    @pl.when(jnp.logical_and(nxt == WIN, w + 1 < nwin))
                    def _():
                        pltpu.make_async_copy(
                            edg_hbm.at[pl.ds(c0 + (w + 1) * WIN, WIN)],
                            idxb[1 - q], semi[1 - q]).wait()
                        pltpu.async_copy(x_hbm.at[idxb[1 - q].at[0, 0]],
                                         rows[1 - t], sems[1 - t])

                    pltpu.sync_copy(ones_v, deg_sh.at[idxb[q].at[lj, 1]],
                                    add=True)
                    pltpu.make_async_copy(x_hbm.at[idxb[q].at[lj, 0]],
                                          rows[t], sems[t]).wait()
                    pltpu.sync_copy(rows[t], agg_sh.at[idxb[q].at[lj, 1]],
                                    add=True)
                return carry
            lax.fori_loop(0, WIN // 2, body, 0)

        def wpair(w2, carry):
            window(w2 * 2, 0)
            window(w2 * 2 + 1, 1)
            return carry
        lax.fori_loop(0, nwin // 2, wpair, 0)
        if nwin % 2:
            window(nwin - 1, 0)

        if extra:
            @pl.when(wid < extra)
            def _():
                tq = nwin % 2
                pltpu.sync_copy(edg_hbm.at[pl.ds(c0 + base, 1)],
                                idxb[tq].at[pl.ds(0, 1)])
                pltpu.async_copy(x_hbm.at[idxb[tq].at[0, 0]],
                                 rows0_v, sem0).wait()
                pltpu.sync_copy(rows0_v, agg_sh.at[idxb[tq].at[0, 1]],
                                add=True)
                pltpu.sync_copy(ones_v, deg_sh.at[idxb[tq].at[0, 1]],
                                add=True)

        plsc.subcore_barrier()

        pltpu.sync_copy(agg_sh.at[pl.ds(row0, ROWS_PER_SUB)],
                        agg_out.at[c, pl.ds(row0, ROWS_PER_SUB)])
        pltpu.sync_copy(deg_sh.at[pl.ds(row0, ROWS_PER_SUB)],
                        deg_out.at[c, pl.ds(row0, ROWS_PER_SUB)])

    return k(x, edg)


def _tc_finale(agg, degb, x, W, b, gamma, beta):
    def body(agg_ref, deg_ref, x_ref, w_ref, b_ref, g_ref, be_ref, o_ref):
        av = agg_ref[...]
        a = av[0] + av[1]
        db = deg_ref[...]
        recip = 1.0 / jnp.maximum(db[:, 0] + db[:, 1], 1.0)
        rt = jnp.transpose(recip)
        m = jnp.concatenate(
            [a[i * 128:(i + 1) * 128] * lax.slice(rt, (0, i), (128, i + 1))
             for i in range(NB)], axis=0)
        o = jnp.dot(m, w_ref[...], preferred_element_type=jnp.float32)
        o = o[:N] + b_ref[...]
        mu = jnp.mean(o, axis=0, keepdims=True)
        var = jnp.mean((o - mu) * (o - mu), axis=0, keepdims=True)
        o = (o - mu) * lax.rsqrt(var + 1e-5) * g_ref[...] + be_ref[...]
        o = jnp.where(o > 0.0, o, jnp.exp(jnp.minimum(o, 0.0)) - 1.0)
        o_ref[...] = x_ref[...] + o

    return pl.pallas_call(
        body,
        out_shape=jax.ShapeDtypeStruct((N, D), jnp.float32),
    )(agg, degb, x, W, b, gamma, beta)


def kernel(x, edges, W, b, gamma, beta):
    E = edges.shape[1]
    assert E % CHUNK == 0
    nchunks = E // CHUNK
    base = nchunks // NW
    base -= base % WIN
    extra = nchunks - base * NW
    assert 0 <= extra < NW or base == 0

    edg = edges.astype(jnp.int32).reshape(2, nchunks, CHUNK)
    edg = edg.transpose(1, 0, 2)

    agg, deg = _sc_segment_sum(x, edg, base, extra)

    degb = deg.reshape(NC, NB, 128).transpose(1, 0, 2)

    return _tc_finale(
        agg, degb, x, W,
        b.reshape(1, D), gamma.reshape(1, D), beta.reshape(1, D),
    )
